# baseline (speedup 1.0000x reference)
import functools

import jax
import jax.numpy as jnp
from jax import lax
from jax.experimental import pallas as pl
from jax.experimental.pallas import tpu as pltpu
from jax.experimental.pallas import tpu_sc as plsc

_LANES = 16
_NW = 32
_CHUNK = 16
_RING = 6
_AHEAD = 4


@jax.jit
def _emb_lookup(x, tokens_embed, positions_embed):
    batch, seq_len = x.shape
    _, d = tokens_embed.shape
    s_blk = seq_len // _NW
    per_b = s_blk // _CHUNK
    n_chunks = batch * per_b

    mesh = plsc.VectorSubcoreMesh(core_axis_name="c", subcore_axis_name="s")

    @functools.partial(
        pl.kernel,
        out_type=jax.ShapeDtypeStruct((batch, seq_len, d), jnp.float32),
        mesh=mesh,
        scratch_types=[
            pltpu.VMEM((batch, s_blk), jnp.int32),
            pltpu.VMEM((_RING, _CHUNK, d), jnp.float32),
            pltpu.VMEM((s_blk, d), jnp.float32),
        ] + [pltpu.SemaphoreType.DMA] * (2 * _RING + 2),
    )
    def emb_kernel(x_hbm, tok_hbm, pos_hbm, out_hbm, idx_v, tokbuf, posbuf,
                   *sems):
        gsem = sems[:_RING]
        osem = sems[_RING:2 * _RING]
        psem, isem = sems[2 * _RING:]
        wid = lax.axis_index("s") * 2 + lax.axis_index("c")
        s0 = wid * s_blk
        icps = [pltpu.async_copy(x_hbm.at[b, pl.ds(s0, s_blk)],
                                 idx_v.at[b], isem)
                for b in range(batch)]
        pcp = pltpu.async_copy(pos_hbm.at[pl.ds(s0, s_blk)], posbuf, psem)

        def for_slot(cb, fn):
            for k in range(_RING):
                @pl.when(cb == k)
                def _():
                    fn(k)

        def issue_gather(c, cb):
            b = c // per_b
            q = lax.rem(c, per_b)
            src = tok_hbm.at[idx_v.at[b, pl.ds(q * _CHUNK, _CHUNK)]]
            for_slot(cb, lambda k: pltpu.async_copy(
                src, tokbuf.at[k], gsem[k]))

        def wait_gather(cb):
            for_slot(cb, lambda k: pltpu.make_async_copy(
                tok_hbm.at[idx_v.at[0, pl.ds(0, _CHUNK)]],
                tokbuf.at[k], gsem[k]).wait())

        def issue_drain(c, cb):
            b = c // per_b
            q = lax.rem(c, per_b)
            dst = out_hbm.at[b, pl.ds(s0 + q * _CHUNK, _CHUNK)]
            for_slot(cb, lambda k: pltpu.async_copy(
                tokbuf.at[k], dst, osem[k]))

        def wait_drain(cb):
            for_slot(cb, lambda k: pltpu.make_async_copy(
                tokbuf.at[k], out_hbm.at[0, pl.ds(s0, _CHUNK)],
                osem[k]).wait())

        for icp in icps:
            icp.wait()
        for c0 in range(_AHEAD):
            issue_gather(jnp.int32(c0), jnp.int32(c0))
        pcp.wait()

        def body(c, _):
            cb = lax.rem(c, _RING)
            q = lax.rem(c, per_b)

            @pl.when(c + _AHEAD < n_chunks)
            def _():
                nb = lax.rem(c + _AHEAD, _RING)

                @pl.when(c + _AHEAD >= _RING)
                def _():
                    wait_drain(nb)
                issue_gather(c + _AHEAD, nb)

            wait_gather(cb)

            @plsc.parallel_loop(0, _CHUNK)
            def _(r):
                for j in range(d // _LANES):
                    sl = pl.ds(j * _LANES, _LANES)
                    tokbuf[cb, r, sl] += posbuf[q * _CHUNK + r, sl]

            issue_drain(c, cb)
            return None

        lax.fori_loop(0, n_chunks, body, None)
        for c in range(n_chunks - _RING, n_chunks):
            wait_drain(jnp.int32(c % _RING))

    return emb_kernel(x, tokens_embed, positions_embed)


def kernel(x, tokens_embed, positions_embed):
    return _emb_lookup(x.astype(jnp.int32), tokens_embed, positions_embed)


# speedup vs baseline: 1.0268x; 1.0268x over previous
import functools

import jax
import jax.numpy as jnp
from jax import lax
from jax.experimental import pallas as pl
from jax.experimental.pallas import tpu as pltpu
from jax.experimental.pallas import tpu_sc as plsc

_LANES = 16
_NW = 32
_CHUNK = 16
_RING = 4
_AHEAD = 2


@jax.jit
def _emb_lookup(x, tokens_embed, positions_embed):
    batch, seq_len = x.shape
    _, d = tokens_embed.shape
    s_blk = seq_len // _NW
    per_b = s_blk // _CHUNK
    n_chunks = batch * per_b

    mesh = plsc.VectorSubcoreMesh(core_axis_name="c", subcore_axis_name="s")

    @functools.partial(
        pl.kernel,
        out_type=jax.ShapeDtypeStruct((batch, seq_len, d), jnp.float32),
        mesh=mesh,
        scratch_types=[
            pltpu.VMEM((batch, s_blk), jnp.int32),
            pltpu.VMEM((_RING, _CHUNK, d), jnp.float32),
            pltpu.VMEM((s_blk, d), jnp.float32),
        ] + [pltpu.SemaphoreType.DMA] * (2 * _RING + 3),
    )
    def emb_kernel(x_hbm, tok_hbm, pos_hbm, out_hbm, idx_v, tokbuf, posbuf,
                   *sems):
        gsem = sems[:_RING]
        osem = sems[_RING:2 * _RING]
        psem, isem, i0sem = sems[2 * _RING:]
        wid = lax.axis_index("s") * 2 + lax.axis_index("c")
        s0 = wid * s_blk
        icps = [pltpu.async_copy(x_hbm.at[b, pl.ds(s0, s_blk)],
                                 idx_v.at[b], i0sem if b == 0 else isem)
                for b in range(batch)]
        pcp = pltpu.async_copy(pos_hbm.at[pl.ds(s0, s_blk)], posbuf, psem)

        def for_slot(cb, fn):
            for k in range(_RING):
                @pl.when(cb == k)
                def _():
                    fn(k)

        def issue_gather(c, cb):
            b = c // per_b
            q = lax.rem(c, per_b)
            src = tok_hbm.at[idx_v.at[b, pl.ds(q * _CHUNK, _CHUNK)]]
            for_slot(cb, lambda k: pltpu.async_copy(
                src, tokbuf.at[k], gsem[k]))

        def wait_gather(cb):
            for_slot(cb, lambda k: pltpu.make_async_copy(
                tok_hbm.at[idx_v.at[0, pl.ds(0, _CHUNK)]],
                tokbuf.at[k], gsem[k]).wait())

        def issue_drain(c, cb):
            b = c // per_b
            q = lax.rem(c, per_b)
            dst = out_hbm.at[b, pl.ds(s0 + q * _CHUNK, _CHUNK)]
            for_slot(cb, lambda k: pltpu.async_copy(
                tokbuf.at[k], dst, osem[k]))

        def wait_drain(cb):
            for_slot(cb, lambda k: pltpu.make_async_copy(
                tokbuf.at[k], out_hbm.at[0, pl.ds(s0, _CHUNK)],
                osem[k]).wait())

        assert _AHEAD <= s_blk // _CHUNK
        icps[0].wait()
        for c0 in range(_AHEAD):
            issue_gather(jnp.int32(c0), jnp.int32(c0))
        for icp in icps[1:]:
            icp.wait()
        pcp.wait()

        def body(c, _):
            cb = lax.rem(c, _RING)
            q = lax.rem(c, per_b)

            @pl.when(c + _AHEAD < n_chunks)
            def _():
                nb = lax.rem(c + _AHEAD, _RING)

                @pl.when(c + _AHEAD >= _RING)
                def _():
                    wait_drain(nb)
                issue_gather(c + _AHEAD, nb)

            wait_gather(cb)

            @plsc.parallel_loop(0, _CHUNK)
            def _(r):
                for j in range(d // _LANES):
                    sl = pl.ds(j * _LANES, _LANES)
                    tokbuf[cb, r, sl] += posbuf[q * _CHUNK + r, sl]

            issue_drain(c, cb)
            return None

        lax.fori_loop(0, n_chunks, body, None)
        for c in range(n_chunks - _RING, n_chunks):
            wait_drain(jnp.int32(c % _RING))

    return emb_kernel(x, tokens_embed, positions_embed)


def kernel(x, tokens_embed, positions_embed):
    return _emb_lookup(x.astype(jnp.int32), tokens_embed, positions_embed)
